# Initial kernel scaffold; baseline (speedup 1.0000x reference)
import jax
import jax.numpy as jnp
from jax.experimental import pallas as pl


def kernel(x, time_deltas, pe):
    raise NotImplementedError("write your pallas kernel here")


# speedup vs baseline: 2.6399x; 2.6399x over previous
import functools

import jax
import jax.numpy as jnp
from jax import lax
from jax.experimental import pallas as pl
from jax.experimental.pallas import tpu as pltpu
from jax.experimental.pallas import tpu_sc as plsc

B, S, D, MAX_LEN = 4096, 200, 64, 512
SD = S * D
L = 16
N_CHUNK = (S + L - 1) // L
S_PAD = N_CHUNK * L
NC, NS = 2, 16
NW = NC * NS
ROWS_PER_W = B // NW


def _pe_body(x_hbm, td_hbm, pe_hbm, out_hbm, pe_v, td_v, cum_v, idx_v, x_v):
    wid = lax.axis_index("s") * NC + lax.axis_index("c")

    pltpu.sync_copy(pe_hbm, pe_v)

    td_v[pl.ds(192, L)] = jnp.zeros((L,), jnp.float32)

    col = [lax.iota(jnp.int32, L) + (j * L) for j in range(D // L)]

    def row_body(i, _):
        b = wid * ROWS_PER_W + i
        pltpu.sync_copy(
            td_hbm.at[pl.ds(pl.multiple_of(b * S, 8), S)], td_v.at[pl.ds(0, S)]
        )

        carry = jnp.zeros((L,), jnp.float32)
        mx = jnp.full((L,), -jnp.inf, jnp.float32)
        for k in range(N_CHUNK):
            v = td_v[pl.ds(k * L, L)]
            cum = plsc.cumsum(v) + carry
            cum_v[pl.ds(k * L, L)] = cum
            carry = carry + jnp.sum(v)
            mx = jnp.maximum(mx, jnp.max(cum))

        dv = mx + jnp.float32(1e-8)
        for k in range(N_CHUNK):
            c = cum_v[pl.ds(k * L, L)]
            t = c / dv * jnp.float32(S)
            iv = jnp.clip(t.astype(jnp.int32), 0, MAX_LEN - 1)
            idx_v[pl.ds(k * L, L)] = iv

        xoff = pl.multiple_of(b * SD, 8)
        pltpu.sync_copy(x_hbm.at[pl.ds(xoff, SD)], x_v.at[pl.ds(0, SD)])

        def chunk_body(k, _c):
            idxvec = idx_v[pl.ds(k * L, L)] * D
            base = pl.multiple_of(k * (L * D), L * D)
            for lane in range(L):
                rv = jnp.full((L,), idxvec[lane], jnp.int32)
                for j in range(D // L):
                    pe_row = plsc.load_gather(pe_v, [rv + col[j]])
                    off = base + lane * D + j * L
                    x_v[pl.ds(off, L)] = x_v[pl.ds(off, L)] + pe_row
            return 0

        lax.fori_loop(0, N_CHUNK, chunk_body, 0)
        pltpu.sync_copy(x_v.at[pl.ds(0, SD)], out_hbm.at[pl.ds(xoff, SD)])
        return 0

    lax.fori_loop(0, ROWS_PER_W, row_body, 0)


_pe_kernel = functools.partial(
    pl.kernel,
    out_type=jax.ShapeDtypeStruct((B * S * D,), jnp.float32),
    mesh=plsc.VectorSubcoreMesh(
        core_axis_name="c", subcore_axis_name="s", num_cores=NC, num_subcores=NS
    ),
    scratch_types=[
        pltpu.VMEM((MAX_LEN * D,), jnp.float32),
        pltpu.VMEM((S_PAD,), jnp.float32),
        pltpu.VMEM((S_PAD,), jnp.float32),
        pltpu.VMEM((S_PAD,), jnp.int32),
        pltpu.VMEM((S_PAD * D,), jnp.float32),
    ],
    compiler_params=pltpu.CompilerParams(needs_layout_passes=False),
)(_pe_body)


def kernel(x, time_deltas, pe):
    out_flat = _pe_kernel(
        x.reshape(B * S * D), time_deltas.reshape(B * S), pe.reshape(MAX_LEN * D)
    )
    return out_flat.reshape(B, S, D)


# speedup vs baseline: 3.0134x; 1.1415x over previous
import functools

import jax
import jax.numpy as jnp
from jax import lax
from jax.experimental import pallas as pl
from jax.experimental.pallas import tpu as pltpu
from jax.experimental.pallas import tpu_sc as plsc

B, S, D, MAX_LEN = 4096, 200, 64, 512
SD = S * D
L = 16
N_CHUNK = (S + L - 1) // L
S_PAD = N_CHUNK * L
NC, NS = 2, 16
NW = NC * NS
ROWS_PER_W = B // NW


def _pe_body(x_hbm, td_hbm, pe_hbm, out_hbm, pe_v, td_v0, td_v1, cum_v, idx_v,
             x_v0, x_v1, sem_td0, sem_td1, sem_xin0, sem_xin1, sem_out0,
             sem_out1):
    td_vs, x_vs = (td_v0, td_v1), (x_v0, x_v1)
    sem_tds, sem_xins, sem_outs = (sem_td0, sem_td1), (sem_xin0, sem_xin1), (sem_out0, sem_out1)
    wid = lax.axis_index("s") * NC + lax.axis_index("c")
    row0 = wid * ROWS_PER_W

    pltpu.sync_copy(pe_hbm, pe_v)

    for slot in range(2):
        td_vs[slot][pl.ds(192, L)] = jnp.zeros((L,), jnp.float32)

    col = [lax.iota(jnp.int32, L) + (j * L) for j in range(D // L)]

    def td_copy(g, slot):
        off = pl.multiple_of((row0 + g) * S, 8)
        return pltpu.make_async_copy(
            td_hbm.at[pl.ds(off, S)], td_vs[slot].at[pl.ds(0, S)], sem_tds[slot]
        )

    def x_in_copy(g, slot):
        off = pl.multiple_of((row0 + g) * SD, 8)
        return pltpu.make_async_copy(
            x_hbm.at[pl.ds(off, SD)], x_vs[slot].at[pl.ds(0, SD)], sem_xins[slot]
        )

    def x_out_copy(g, slot):
        off = pl.multiple_of((row0 + g) * SD, 8)
        return pltpu.make_async_copy(
            x_vs[slot].at[pl.ds(0, SD)], out_hbm.at[pl.ds(off, SD)], sem_outs[slot]
        )

    td_copy(0, 0).start()
    x_in_copy(0, 0).start()

    def process_row(g, slot):
        nslot = 1 - slot
        td_copy(g, slot).wait()

        @pl.when(g >= 1)
        def _():
            x_out_copy(g - 1, nslot).wait()

        @pl.when(g + 1 < ROWS_PER_W)
        def _():
            td_copy(g + 1, nslot).start()
            x_in_copy(g + 1, nslot).start()

        carry = jnp.zeros((L,), jnp.float32)
        mx = jnp.full((L,), -jnp.inf, jnp.float32)
        for k in range(N_CHUNK):
            v = td_vs[slot][pl.ds(k * L, L)]
            cum = plsc.cumsum(v) + carry
            cum_v[pl.ds(k * L, L)] = cum
            carry = carry + jnp.sum(v)
            mx = jnp.maximum(mx, jnp.max(cum))

        dv = mx + jnp.float32(1e-8)
        for k in range(N_CHUNK):
            c = cum_v[pl.ds(k * L, L)]
            t = c / dv * jnp.float32(S)
            iv = jnp.clip(t.astype(jnp.int32), 0, MAX_LEN - 1)
            idx_v[pl.ds(k * L, L)] = iv

        x_in_copy(g, slot).wait()

        def chunk_body(k, _c):
            idxvec = idx_v[pl.ds(k * L, L)] * D
            base = pl.multiple_of(k * (L * D), L * D)
            for lane in range(L):
                rv = jnp.full((L,), idxvec[lane], jnp.int32)
                for j in range(D // L):
                    pe_row = plsc.load_gather(pe_v, [rv + col[j]])
                    off = base + lane * D + j * L
                    x_vs[slot][pl.ds(off, L)] = x_vs[slot][pl.ds(off, L)] + pe_row
            return 0

        lax.fori_loop(0, N_CHUNK, chunk_body, 0)
        x_out_copy(g, slot).start()

    def pair_body(t, _):
        process_row(t * 2, 0)
        process_row(t * 2 + 1, 1)
        return 0

    lax.fori_loop(0, ROWS_PER_W // 2, pair_body, 0)
    x_out_copy(ROWS_PER_W - 1, 1).wait()


_pe_kernel = functools.partial(
    pl.kernel,
    out_type=jax.ShapeDtypeStruct((B * S * D,), jnp.float32),
    mesh=plsc.VectorSubcoreMesh(
        core_axis_name="c", subcore_axis_name="s", num_cores=NC, num_subcores=NS
    ),
    scratch_types=[
        pltpu.VMEM((MAX_LEN * D,), jnp.float32),
        pltpu.VMEM((S_PAD,), jnp.float32),
        pltpu.VMEM((S_PAD,), jnp.float32),
        pltpu.VMEM((S_PAD,), jnp.float32),
        pltpu.VMEM((S_PAD,), jnp.int32),
        pltpu.VMEM((S_PAD * D,), jnp.float32),
        pltpu.VMEM((S_PAD * D,), jnp.float32),
        pltpu.SemaphoreType.DMA,
        pltpu.SemaphoreType.DMA,
        pltpu.SemaphoreType.DMA,
        pltpu.SemaphoreType.DMA,
        pltpu.SemaphoreType.DMA,
        pltpu.SemaphoreType.DMA,
    ],
    compiler_params=pltpu.CompilerParams(needs_layout_passes=False),
)(_pe_body)


def kernel(x, time_deltas, pe):
    out_flat = _pe_kernel(
        x.reshape(B * S * D), time_deltas.reshape(B * S), pe.reshape(MAX_LEN * D)
    )
    return out_flat.reshape(B, S, D)
